# baseline (speedup 1.0000x reference)
import functools

import jax
import jax.numpy as jnp
from jax import lax
from jax.experimental import pallas as pl
from jax.experimental.pallas import tpu as pltpu
from jax.experimental.pallas import tpu_sc as plsc

N = 10000
D = 128
E = 320000
NC = 2
NS = 16
NW = NC * NS
CHUNK = 128
CPW = 80
E_PAD = NW * CPW * CHUNK
RPW = 624
TAIL = N - NS * RPW
BR = 1000
GRID = N // BR


_sc_mesh = plsc.VectorSubcoreMesh(core_axis_name="c", subcore_axis_name="s")


@functools.partial(
    pl.kernel,
    out_type=jax.ShapeDtypeStruct((2 * N, D), jnp.float32),
    mesh=_sc_mesh,
    scratch_types=[
        pltpu.VMEM_SHARED((N + 16, D), jnp.float32),
        pltpu.VMEM((CHUNK,), jnp.int32),
        pltpu.VMEM((CHUNK,), jnp.int32),
        pltpu.VMEM((CHUNK,), jnp.int32),
        pltpu.VMEM((CHUNK,), jnp.int32),
        pltpu.VMEM((CHUNK, D), jnp.float32),
        pltpu.SemaphoreType.DMA,
        pltpu.SemaphoreType.DMA,
    ],
)
def _sc_edge(m_hbm, srcp_hbm, dstp_hbm, z_hbm, out_hbm, acc, sidxa, didxa,
             sidxb, didxb, rows, gsem, isem):
    c = lax.axis_index("c")
    s = lax.axis_index("s")
    wid = c * NS + s

    pltpu.sync_copy(z_hbm, acc.at[pl.ds(s * RPW, RPW)])

    @pl.when(s == NS - 1)
    def _zero_tail():
        pltpu.sync_copy(z_hbm.at[pl.ds(0, TAIL)], acc.at[pl.ds(NS * RPW, TAIL)])

    plsc.subcore_barrier()

    def start_idx(j, sb, db):
        base = pl.multiple_of((wid * CPW + j) * CHUNK, CHUNK)
        h1 = pltpu.async_copy(srcp_hbm.at[pl.ds(base, CHUNK)], sb, isem)
        h2 = pltpu.async_copy(dstp_hbm.at[pl.ds(base, CHUNK)], db, isem)
        return h1, h2

    def body(g, carry):
        j0 = g * 2
        h1, h2 = start_idx(j0, sidxa, didxa)
        h1.wait()
        h2.wait()
        pltpu.async_copy(m_hbm.at[sidxa], rows, gsem).wait()
        h3, h4 = start_idx(j0 + 1, sidxb, didxb)
        pltpu.sync_copy(rows, acc.at[didxa], add=True)
        h3.wait()
        h4.wait()
        pltpu.async_copy(m_hbm.at[sidxb], rows, gsem).wait()
        pltpu.sync_copy(rows, acc.at[didxb], add=True)
        return carry

    lax.fori_loop(0, CPW // 2, body, 0, unroll=False)
    plsc.subcore_barrier()

    out_base = c * N + s * RPW
    pltpu.sync_copy(acc.at[pl.ds(s * RPW, RPW)], out_hbm.at[pl.ds(out_base, RPW)])

    @pl.when(s == NS - 1)
    def _write_tail():
        pltpu.sync_copy(acc.at[pl.ds(NS * RPW, TAIL)],
                        out_hbm.at[pl.ds(c * N + NS * RPW, TAIL)])


def _mm_body(x_ref, wt_ref, b_ref, o_ref):
    o_ref[...] = (
        jnp.dot(x_ref[...], wt_ref[...], preferred_element_type=jnp.float32)
        + b_ref[...]
    )


def _mm_bias(x, wt, b):
    return pl.pallas_call(
        _mm_body,
        grid=(GRID,),
        in_specs=[
            pl.BlockSpec((BR, D), lambda i: (i, 0)),
            pl.BlockSpec(wt.shape, lambda i: (0, 0)),
            pl.BlockSpec((1, wt.shape[1]), lambda i: (0, 0)),
        ],
        out_specs=pl.BlockSpec((BR, wt.shape[1]), lambda i: (i, 0)),
        out_shape=jax.ShapeDtypeStruct((N, wt.shape[1]), jnp.float32),
    )(x, wt, b)


def _gru_body(ap0, ap1, h_ref, wih, whh, bih, bhh, wn, wbn, ho, mo):
    a = ap0[...] + ap1[...]
    h = h_ref[...]
    gi = jnp.dot(a, wih[...], preferred_element_type=jnp.float32) + bih[...]
    gh = jnp.dot(h, whh[...], preferred_element_type=jnp.float32) + bhh[...]
    r = jax.nn.sigmoid(gi[:, :D] + gh[:, :D])
    z = jax.nn.sigmoid(gi[:, D:2 * D] + gh[:, D:2 * D])
    n = jnp.tanh(gi[:, 2 * D:] + r * gh[:, 2 * D:])
    hn = (1.0 - z) * n + z * h
    ho[...] = hn
    mo[...] = (
        jnp.dot(hn, wn[...], preferred_element_type=jnp.float32) + wbn[...]
    )


def _gru_step(ap, h, wihT, whhT, bih, bhh, wnT, wbn):
    return pl.pallas_call(
        _gru_body,
        grid=(GRID,),
        in_specs=[
            pl.BlockSpec((BR, D), lambda i: (i, 0)),
            pl.BlockSpec((BR, D), lambda i: (i + GRID, 0)),
            pl.BlockSpec((BR, D), lambda i: (i, 0)),
            pl.BlockSpec((D, 3 * D), lambda i: (0, 0)),
            pl.BlockSpec((D, 3 * D), lambda i: (0, 0)),
            pl.BlockSpec((1, 3 * D), lambda i: (0, 0)),
            pl.BlockSpec((1, 3 * D), lambda i: (0, 0)),
            pl.BlockSpec((D, D), lambda i: (0, 0)),
            pl.BlockSpec((1, D), lambda i: (0, 0)),
        ],
        out_specs=[
            pl.BlockSpec((BR, D), lambda i: (i, 0)),
            pl.BlockSpec((BR, D), lambda i: (i, 0)),
        ],
        out_shape=[
            jax.ShapeDtypeStruct((N, D), jnp.float32),
            jax.ShapeDtypeStruct((N, D), jnp.float32),
        ],
    )(ap, ap, h, wihT, whhT, bih, bhh, wnT, wbn)


def _colsum_body(h_ref, o_ref):
    @pl.when(pl.program_id(0) == 0)
    def _init():
        o_ref[...] = jnp.zeros_like(o_ref)

    o_ref[...] += jnp.sum(h_ref[...], axis=0, keepdims=True)


def _colsum(h):
    return pl.pallas_call(
        _colsum_body,
        grid=(GRID,),
        in_specs=[pl.BlockSpec((BR, D), lambda i: (i, 0))],
        out_specs=pl.BlockSpec((1, D), lambda i: (0, 0)),
        out_shape=jax.ShapeDtypeStruct((1, D), jnp.float32),
    )(h)


def _head_body(s_ref, wct_ref, bc_ref, o_ref):
    o_ref[...] = (
        jnp.dot(s_ref[...] * (1.0 / N), wct_ref[...],
                preferred_element_type=jnp.float32)
        + bc_ref[...]
    )


def _head(s, wcT, bc):
    k = wcT.shape[1]
    return pl.pallas_call(
        _head_body,
        in_specs=[
            pl.BlockSpec((1, D), lambda: (0, 0)),
            pl.BlockSpec((D, k), lambda: (0, 0)),
            pl.BlockSpec((1, k), lambda: (0, 0)),
        ],
        out_specs=pl.BlockSpec((1, k), lambda: (0, 0)),
        out_shape=jax.ShapeDtypeStruct((1, k), jnp.float32),
    )(s, wcT, bc)


def kernel(x, edge_index, W0, Wb0, Wih0, Whh0, bih0, bhh0,
           W1, Wb1, Wih1, Whh1, bih1, bhh1, Wc, bc):
    pad = E_PAD - E
    srcp = jnp.concatenate(
        [edge_index[0], jnp.zeros((pad,), jnp.int32)]
    )
    dstp = jnp.concatenate(
        [edge_index[1], jnp.full((pad,), N, jnp.int32)]
    )
    z = jnp.zeros((RPW, D), jnp.float32)

    WT = [W0.T, W1.T]
    Wb = [Wb0.reshape(1, D), Wb1.reshape(1, D)]
    gru_params = [
        (Wih0.T, Whh0.T, bih0.reshape(1, 3 * D), bhh0.reshape(1, 3 * D)),
        (Wih1.T, Whh1.T, bih1.reshape(1, 3 * D), bhh1.reshape(1, 3 * D)),
    ]

    h = x
    m = _mm_bias(h, WT[0], Wb[0])
    for k in range(6):
        layer = k // 3
        nxt = min((k + 1) // 3, 1)
        ap = _sc_edge(m, srcp, dstp, z)
        h, m = _gru_step(ap, h, *gru_params[layer], WT[nxt], Wb[nxt])

    return _head(_colsum(h), Wc.T, bc.reshape(1, 16))


# speedup vs baseline: 1.5321x; 1.5321x over previous
import functools

import jax
import jax.numpy as jnp
from jax import lax
from jax.experimental import pallas as pl
from jax.experimental.pallas import tpu as pltpu
from jax.experimental.pallas import tpu_sc as plsc

N = 10000
D = 128
E = 320000
NC = 2
NS = 16
NW = NC * NS
CHUNK = 128
CPW = 79
E_PAD = NW * CPW * CHUNK
RPW = 624
TAIL = N - NS * RPW
BR = 1000
GRID = N // BR


_sc_mesh = plsc.VectorSubcoreMesh(core_axis_name="c", subcore_axis_name="s")


@functools.partial(
    pl.kernel,
    out_type=jax.ShapeDtypeStruct((2 * N, D), jnp.float32),
    mesh=_sc_mesh,
    scratch_types=[
        pltpu.VMEM_SHARED((N + 16, D), jnp.float32),
        pltpu.VMEM((CHUNK,), jnp.int32),
        pltpu.VMEM((CHUNK,), jnp.int32),
        pltpu.VMEM((CHUNK, D), jnp.float32),
        pltpu.SemaphoreType.DMA,
        pltpu.SemaphoreType.DMA,
    ],
)
def _sc_edge(m_hbm, srcp_hbm, dstp_hbm, z_hbm, out_hbm, acc, sidx, didx, rows,
             gsem, isem):
    c = lax.axis_index("c")
    s = lax.axis_index("s")
    wid = c * NS + s

    pltpu.sync_copy(z_hbm, acc.at[pl.ds(s * RPW, RPW)])

    @pl.when(s == NS - 1)
    def _zero_tail():
        pltpu.sync_copy(z_hbm.at[pl.ds(0, TAIL)], acc.at[pl.ds(NS * RPW, TAIL)])

    plsc.subcore_barrier()

    def body(j, carry):
        base = pl.multiple_of((wid * CPW + j) * CHUNK, CHUNK)
        h1 = pltpu.async_copy(srcp_hbm.at[pl.ds(base, CHUNK)], sidx, isem)
        h2 = pltpu.async_copy(dstp_hbm.at[pl.ds(base, CHUNK)], didx, isem)
        h1.wait()
        h2.wait()
        pltpu.async_copy(m_hbm.at[sidx], rows, gsem).wait()
        pltpu.sync_copy(rows, acc.at[didx], add=True)
        return carry

    lax.fori_loop(0, CPW, body, 0, unroll=False)
    plsc.subcore_barrier()

    out_base = c * N + s * RPW
    pltpu.sync_copy(acc.at[pl.ds(s * RPW, RPW)], out_hbm.at[pl.ds(out_base, RPW)])

    @pl.when(s == NS - 1)
    def _write_tail():
        pltpu.sync_copy(acc.at[pl.ds(NS * RPW, TAIL)],
                        out_hbm.at[pl.ds(c * N + NS * RPW, TAIL)])


def _mm_body(x_ref, wt_ref, b_ref, o_ref):
    o_ref[...] = (
        jnp.dot(x_ref[...], wt_ref[...], preferred_element_type=jnp.float32)
        + b_ref[...]
    )


def _mm_bias(x, wt, b):
    return pl.pallas_call(
        _mm_body,
        grid=(GRID,),
        in_specs=[
            pl.BlockSpec((BR, D), lambda i: (i, 0)),
            pl.BlockSpec(wt.shape, lambda i: (0, 0)),
            pl.BlockSpec((1, wt.shape[1]), lambda i: (0, 0)),
        ],
        out_specs=pl.BlockSpec((BR, wt.shape[1]), lambda i: (i, 0)),
        out_shape=jax.ShapeDtypeStruct((N, wt.shape[1]), jnp.float32),
    )(x, wt, b)


def _gru_body(ap0, ap1, h_ref, wih, whh, bih, bhh, wn, wbn, ho, mo):
    a = ap0[...] + ap1[...]
    h = h_ref[...]
    gi = jnp.dot(a, wih[...], preferred_element_type=jnp.float32) + bih[...]
    gh = jnp.dot(h, whh[...], preferred_element_type=jnp.float32) + bhh[...]
    r = jax.nn.sigmoid(gi[:, :D] + gh[:, :D])
    z = jax.nn.sigmoid(gi[:, D:2 * D] + gh[:, D:2 * D])
    n = jnp.tanh(gi[:, 2 * D:] + r * gh[:, 2 * D:])
    hn = (1.0 - z) * n + z * h
    ho[...] = hn
    mo[...] = (
        jnp.dot(hn, wn[...], preferred_element_type=jnp.float32) + wbn[...]
    )


def _gru_step(ap, h, wihT, whhT, bih, bhh, wnT, wbn):
    return pl.pallas_call(
        _gru_body,
        grid=(GRID,),
        in_specs=[
            pl.BlockSpec((BR, D), lambda i: (i, 0)),
            pl.BlockSpec((BR, D), lambda i: (i + GRID, 0)),
            pl.BlockSpec((BR, D), lambda i: (i, 0)),
            pl.BlockSpec((D, 3 * D), lambda i: (0, 0)),
            pl.BlockSpec((D, 3 * D), lambda i: (0, 0)),
            pl.BlockSpec((1, 3 * D), lambda i: (0, 0)),
            pl.BlockSpec((1, 3 * D), lambda i: (0, 0)),
            pl.BlockSpec((D, D), lambda i: (0, 0)),
            pl.BlockSpec((1, D), lambda i: (0, 0)),
        ],
        out_specs=[
            pl.BlockSpec((BR, D), lambda i: (i, 0)),
            pl.BlockSpec((BR, D), lambda i: (i, 0)),
        ],
        out_shape=[
            jax.ShapeDtypeStruct((N, D), jnp.float32),
            jax.ShapeDtypeStruct((N, D), jnp.float32),
        ],
    )(ap, ap, h, wihT, whhT, bih, bhh, wnT, wbn)


def _colsum_body(h_ref, o_ref):
    @pl.when(pl.program_id(0) == 0)
    def _init():
        o_ref[...] = jnp.zeros_like(o_ref)

    o_ref[...] += jnp.sum(h_ref[...], axis=0, keepdims=True)


def _colsum(h):
    return pl.pallas_call(
        _colsum_body,
        grid=(GRID,),
        in_specs=[pl.BlockSpec((BR, D), lambda i: (i, 0))],
        out_specs=pl.BlockSpec((1, D), lambda i: (0, 0)),
        out_shape=jax.ShapeDtypeStruct((1, D), jnp.float32),
    )(h)


def _head_body(s_ref, wct_ref, bc_ref, o_ref):
    o_ref[...] = (
        jnp.dot(s_ref[...] * (1.0 / N), wct_ref[...],
                preferred_element_type=jnp.float32)
        + bc_ref[...]
    )


def _head(s, wcT, bc):
    k = wcT.shape[1]
    return pl.pallas_call(
        _head_body,
        in_specs=[
            pl.BlockSpec((1, D), lambda: (0, 0)),
            pl.BlockSpec((D, k), lambda: (0, 0)),
            pl.BlockSpec((1, k), lambda: (0, 0)),
        ],
        out_specs=pl.BlockSpec((1, k), lambda: (0, 0)),
        out_shape=jax.ShapeDtypeStruct((1, k), jnp.float32),
    )(s, wcT, bc)


def kernel(x, edge_index, W0, Wb0, Wih0, Whh0, bih0, bhh0,
           W1, Wb1, Wih1, Whh1, bih1, bhh1, Wc, bc):
    pad = E_PAD - E
    srcp = jnp.concatenate(
        [edge_index[0], jnp.zeros((pad,), jnp.int32)]
    )
    dstp = jnp.concatenate(
        [edge_index[1], jnp.full((pad,), N, jnp.int32)]
    )
    z = jnp.zeros((RPW, D), jnp.float32)

    WT = [W0.T, W1.T]
    Wb = [Wb0.reshape(1, D), Wb1.reshape(1, D)]
    gru_params = [
        (Wih0.T, Whh0.T, bih0.reshape(1, 3 * D), bhh0.reshape(1, 3 * D)),
        (Wih1.T, Whh1.T, bih1.reshape(1, 3 * D), bhh1.reshape(1, 3 * D)),
    ]

    h = x
    m = _mm_bias(h, WT[0], Wb[0])
    for k in range(6):
        layer = k // 3
        nxt = min((k + 1) // 3, 1)
        ap = _sc_edge(m, srcp, dstp, z)
        h, m = _gru_step(ap, h, *gru_params[layer], WT[nxt], Wb[nxt])

    return _head(_colsum(h), Wc.T, bc.reshape(1, 16))
